# initial kernel scaffold (unmeasured)
import jax
import jax.numpy as jnp
from jax import lax
from jax.experimental import pallas as pl
from jax.experimental.pallas import tpu as pltpu


def kernel(
    x,
):
    def body(*refs):
        pass

    out_shape = jax.ShapeDtypeStruct(..., jnp.float32)
    return pl.pallas_call(body, out_shape=out_shape)(...)


# baseline (device time: 356500 ns/iter reference)
import functools

import jax
import jax.numpy as jnp
from jax import lax
from jax.experimental import pallas as pl
from jax.experimental.pallas import tpu as pltpu

N_DEV = 8
N_HOP = N_DEV - 1


def _slot_to_mesh(s):
    return jnp.where(s < 4, s, 11 - s)


def kernel(x):
    m_per, n = x.shape
    ch = m_per // N_DEV

    def body(x_ref, out_ref, comm_ref, rs_send_sems, rs_recv_sems,
             ag_send_sems, ag_recv_sems):
        me = lax.axis_index("i")
        r = _slot_to_mesh(me)
        succ = _slot_to_mesh(lax.rem(r + 1, N_DEV))
        pred = _slot_to_mesh(lax.rem(r + N_DEV - 1, N_DEV))

        barrier_sem = pltpu.get_barrier_semaphore()
        for nbr in (pred, succ):
            pl.semaphore_signal(
                barrier_sem, inc=1,
                device_id=(nbr,), device_id_type=pl.DeviceIdType.MESH,
            )
        pl.semaphore_wait(barrier_sem, 2)

        out_ref[...] = x_ref[...]

        for s in range(N_HOP):
            send_c = lax.rem(r - s + N_DEV, N_DEV)
            recv_c = lax.rem(r - s - 1 + N_DEV, N_DEV)
            rdma = pltpu.make_async_remote_copy(
                src_ref=out_ref.at[pl.ds(send_c * ch, ch)],
                dst_ref=comm_ref.at[s],
                send_sem=rs_send_sems.at[s],
                recv_sem=rs_recv_sems.at[s],
                device_id=(succ,),
                device_id_type=pl.DeviceIdType.MESH,
            )
            rdma.start()
            rdma.wait()
            idx = pl.ds(recv_c * ch, ch)
            out_ref[idx] = out_ref[idx] + comm_ref[s]

        for s in range(N_HOP):
            c = lax.rem(r + 1 - s + N_DEV, N_DEV)
            idx = pl.ds(c * ch, ch)
            rdma = pltpu.make_async_remote_copy(
                src_ref=out_ref.at[idx],
                dst_ref=out_ref.at[idx],
                send_sem=ag_send_sems.at[s],
                recv_sem=ag_recv_sems.at[s],
                device_id=(succ,),
                device_id_type=pl.DeviceIdType.MESH,
            )
            rdma.start()
            rdma.wait()

        @functools.partial(
            pl.run_scoped, second_barrier=pltpu.SemaphoreType.REGULAR
        )
        def _(second_barrier):
            for nbr in (pred, succ):
                pl.semaphore_signal(
                    second_barrier, inc=1,
                    device_id=(nbr,), device_id_type=pl.DeviceIdType.MESH,
                )
            pl.semaphore_wait(second_barrier, 2)

    return pl.pallas_call(
        body,
        out_shape=jax.ShapeDtypeStruct((m_per, n), x.dtype),
        in_specs=[pl.BlockSpec(memory_space=pltpu.VMEM)],
        out_specs=pl.BlockSpec(memory_space=pltpu.VMEM),
        scratch_shapes=[
            pltpu.VMEM((N_HOP, ch, n), x.dtype),
            pltpu.SemaphoreType.DMA((N_HOP,)),
            pltpu.SemaphoreType.DMA((N_HOP,)),
            pltpu.SemaphoreType.DMA((N_HOP,)),
            pltpu.SemaphoreType.DMA((N_HOP,)),
        ],
        compiler_params=pltpu.CompilerParams(collective_id=0),
    )(x)


# device time: 135770 ns/iter; 2.6258x vs baseline; 2.6258x over previous
import functools

import jax
import jax.numpy as jnp
from jax import lax
from jax.experimental import pallas as pl
from jax.experimental.pallas import tpu as pltpu

N_DEV = 8

_MASK = {"x": 1, "y": 3, "z": 4}

_PARTS = (
    (0, 1408, ("x", "y", "z")),
    (1408, 1344, ("y", "z", "x")),
    (2752, 1344, ("z", "x", "y")),
)


def kernel(x):
    m_per, n = x.shape
    assert m_per == 4096

    def body(x_ref, out_ref, comm_ref, rs_send, rs_recv, ag_send, ag_recv):
        me = lax.axis_index("i")
        partners = {d: me ^ mask for d, mask in _MASK.items()}
        coords = {
            "x": (me ^ (me >> 1)) & 1,
            "y": (me >> 1) & 1,
            "z": (me >> 2) & 1,
        }

        barrier_sem = pltpu.get_barrier_semaphore()
        for d in ("x", "y", "z"):
            pl.semaphore_signal(
                barrier_sem, inc=1,
                device_id=(partners[d],), device_id_type=pl.DeviceIdType.MESH,
            )
        pl.semaphore_wait(barrier_sem, 3)

        out_ref[...] = x_ref[...]

        seg_lo = {}
        halves = {}

        def rs_descr(p, k):
            base, rows, dims = _PARTS[p]
            d = dims[k]
            part = partners[d]
            half = coords[d]
            halves[(p, k)] = half
            l2 = rows >> (k + 1)
            lo = seg_lo[p]
            send_lo = lo + (1 - half) * l2
            off = base + (0 if k == 0 else (rows >> 1) if k == 1 else 3 * (rows >> 2))
            return pltpu.make_async_remote_copy(
                src_ref=out_ref.at[pl.ds(send_lo, l2)],
                dst_ref=comm_ref.at[pl.ds(off, l2)],
                send_sem=rs_send.at[p, k],
                recv_sem=rs_recv.at[p, k],
                device_id=(part,),
                device_id_type=pl.DeviceIdType.MESH,
            ), off, l2

        def ag_descr(p, k):
            base, rows, dims = _PARTS[p]
            d = dims[2 - k]
            part = partners[d]
            l = rows >> (3 - k)
            lo = seg_lo[p]
            return pltpu.make_async_remote_copy(
                src_ref=out_ref.at[pl.ds(lo, l)],
                dst_ref=out_ref.at[pl.ds(lo, l)],
                send_sem=ag_send.at[p, k],
                recv_sem=ag_recv.at[p, k],
                device_id=(part,),
                device_id_type=pl.DeviceIdType.MESH,
            )

        for p in range(3):
            seg_lo[p] = jnp.int32(_PARTS[p][0])
        inflight = {}
        for p in range(3):
            rdma, off, l2 = rs_descr(p, 0)
            rdma.start()
            inflight[p] = (rdma, off, l2)

        ag_inflight = {}
        for k in range(3):
            for p in range(3):
                rdma, off, l2 = inflight[p]
                rdma.wait()
                half = halves[(p, k)]
                keep_lo = seg_lo[p] + half * l2
                idx = pl.ds(keep_lo, l2)
                out_ref[idx] = out_ref[idx] + comm_ref[pl.ds(off, l2)]
                seg_lo[p] = keep_lo
                if k < 2:
                    nxt, noff, nl2 = rs_descr(p, k + 1)
                    nxt.start()
                    inflight[p] = (nxt, noff, nl2)
                else:
                    ag = ag_descr(p, 0)
                    ag.start()
                    ag_inflight[p] = ag

        for k in range(3):
            for p in range(3):
                rows = _PARTS[p][1]
                l = rows >> (3 - k)
                ag_inflight[p].wait()
                seg_lo[p] = seg_lo[p] - halves[(p, 2 - k)] * l
                if k < 2:
                    ag = ag_descr(p, k + 1)
                    ag.start()
                    ag_inflight[p] = ag

        @functools.partial(
            pl.run_scoped, second_barrier=pltpu.SemaphoreType.REGULAR
        )
        def _(second_barrier):
            for d in ("x", "y", "z"):
                pl.semaphore_signal(
                    second_barrier, inc=1,
                    device_id=(partners[d],),
                    device_id_type=pl.DeviceIdType.MESH,
                )
            pl.semaphore_wait(second_barrier, 3)

    return pl.pallas_call(
        body,
        out_shape=jax.ShapeDtypeStruct((m_per, n), x.dtype),
        in_specs=[pl.BlockSpec(memory_space=pltpu.VMEM)],
        out_specs=pl.BlockSpec(memory_space=pltpu.VMEM),
        scratch_shapes=[
            pltpu.VMEM((m_per, n), x.dtype),
            pltpu.SemaphoreType.DMA((3, 3)),
            pltpu.SemaphoreType.DMA((3, 3)),
            pltpu.SemaphoreType.DMA((3, 3)),
            pltpu.SemaphoreType.DMA((3, 3)),
        ],
        compiler_params=pltpu.CompilerParams(collective_id=0),
    )(x)


# device time: 134725 ns/iter; 2.6461x vs baseline; 1.0078x over previous
import functools

import jax
import jax.numpy as jnp
from jax import lax
from jax.experimental import pallas as pl
from jax.experimental.pallas import tpu as pltpu

N_DEV = 8

_MASK = {"x": 1, "y": 3, "z": 4}

_PARTS = (
    (0, 1408, ("x", "y", "z")),
    (1408, 1344, ("y", "z", "x")),
    (2752, 1344, ("z", "x", "y")),
)


def kernel(x):
    m_per, n = x.shape
    assert m_per == 4096

    def body(x_ref, out_ref, comm_ref, rs_send, rs_recv, ag_send, ag_recv):
        me = lax.axis_index("i")
        partners = {d: me ^ mask for d, mask in _MASK.items()}
        coords = {
            "x": (me ^ (me >> 1)) & 1,
            "y": (me >> 1) & 1,
            "z": (me >> 2) & 1,
        }

        barrier_sem = pltpu.get_barrier_semaphore()
        for d in ("x", "y", "z"):
            pl.semaphore_signal(
                barrier_sem, inc=1,
                device_id=(partners[d],), device_id_type=pl.DeviceIdType.MESH,
            )
        pl.semaphore_wait(barrier_sem, 3)


        seg_lo = {}
        halves = {}

        def rs_descr(p, k):
            base, rows, dims = _PARTS[p]
            d = dims[k]
            part = partners[d]
            half = coords[d]
            halves[(p, k)] = half
            l2 = rows >> (k + 1)
            lo = seg_lo[p]
            send_lo = lo + (1 - half) * l2
            off = base + (0 if k == 0 else (rows >> 1) if k == 1 else 3 * (rows >> 2))
            src = x_ref if k == 0 else out_ref
            return pltpu.make_async_remote_copy(
                src_ref=src.at[pl.ds(send_lo, l2)],
                dst_ref=comm_ref.at[pl.ds(off, l2)],
                send_sem=rs_send.at[p, k],
                recv_sem=rs_recv.at[p, k],
                device_id=(part,),
                device_id_type=pl.DeviceIdType.MESH,
            ), off, l2

        def ag_descr(p, k):
            base, rows, dims = _PARTS[p]
            d = dims[2 - k]
            part = partners[d]
            l = rows >> (3 - k)
            lo = seg_lo[p]
            return pltpu.make_async_remote_copy(
                src_ref=out_ref.at[pl.ds(lo, l)],
                dst_ref=out_ref.at[pl.ds(lo, l)],
                send_sem=ag_send.at[p, k],
                recv_sem=ag_recv.at[p, k],
                device_id=(part,),
                device_id_type=pl.DeviceIdType.MESH,
            )

        for p in range(3):
            seg_lo[p] = jnp.int32(_PARTS[p][0])
        inflight = {}
        for p in range(3):
            rdma, off, l2 = rs_descr(p, 0)
            rdma.start()
            inflight[p] = (rdma, off, l2)

        ag_inflight = {}
        for k in range(3):
            for p in range(3):
                rdma, off, l2 = inflight[p]
                rdma.wait()
                half = halves[(p, k)]
                keep_lo = seg_lo[p] + half * l2
                idx = pl.ds(keep_lo, l2)
                acc_src = x_ref if k == 0 else out_ref
                out_ref[idx] = acc_src[idx] + comm_ref[pl.ds(off, l2)]
                seg_lo[p] = keep_lo
                if k < 2:
                    nxt, noff, nl2 = rs_descr(p, k + 1)
                    nxt.start()
                    inflight[p] = (nxt, noff, nl2)
                else:
                    ag = ag_descr(p, 0)
                    ag.start()
                    ag_inflight[p] = ag

        for k in range(3):
            for p in range(3):
                rows = _PARTS[p][1]
                l = rows >> (3 - k)
                ag_inflight[p].wait()
                seg_lo[p] = seg_lo[p] - halves[(p, 2 - k)] * l
                if k < 2:
                    ag = ag_descr(p, k + 1)
                    ag.start()
                    ag_inflight[p] = ag

        @functools.partial(
            pl.run_scoped, second_barrier=pltpu.SemaphoreType.REGULAR
        )
        def _(second_barrier):
            for d in ("x", "y", "z"):
                pl.semaphore_signal(
                    second_barrier, inc=1,
                    device_id=(partners[d],),
                    device_id_type=pl.DeviceIdType.MESH,
                )
            pl.semaphore_wait(second_barrier, 3)

    return pl.pallas_call(
        body,
        out_shape=jax.ShapeDtypeStruct((m_per, n), x.dtype),
        in_specs=[pl.BlockSpec(memory_space=pltpu.VMEM)],
        out_specs=pl.BlockSpec(memory_space=pltpu.VMEM),
        scratch_shapes=[
            pltpu.VMEM((m_per, n), x.dtype),
            pltpu.SemaphoreType.DMA((3, 3)),
            pltpu.SemaphoreType.DMA((3, 3)),
            pltpu.SemaphoreType.DMA((3, 3)),
            pltpu.SemaphoreType.DMA((3, 3)),
        ],
        compiler_params=pltpu.CompilerParams(collective_id=0),
    )(x)


# device time: 129791 ns/iter; 2.7467x vs baseline; 1.0380x over previous
import functools

import jax
import jax.numpy as jnp
from jax import lax
from jax.experimental import pallas as pl
from jax.experimental.pallas import tpu as pltpu

N_DEV = 8

_MASK = {"x": 1, "y": 3, "z": 4}

_PARTS = (
    (0, 704, ("x", "y", "z")),
    (704, 704, ("x", "y", "z")),
    (1408, 704, ("y", "z", "x")),
    (2112, 640, ("y", "z", "x")),
    (2752, 704, ("z", "x", "y")),
    (3456, 640, ("z", "x", "y")),
)
_NP = len(_PARTS)


def kernel(x):
    m_per, n = x.shape
    assert m_per == 4096

    def body(x_ref, out_ref, comm_ref, rs_send, rs_recv, ag_send, ag_recv):
        me = lax.axis_index("i")
        partners = {d: me ^ mask for d, mask in _MASK.items()}
        coords = {
            "x": (me ^ (me >> 1)) & 1,
            "y": (me >> 1) & 1,
            "z": (me >> 2) & 1,
        }

        barrier_sem = pltpu.get_barrier_semaphore()
        for d in ("x", "y", "z"):
            pl.semaphore_signal(
                barrier_sem, inc=1,
                device_id=(partners[d],), device_id_type=pl.DeviceIdType.MESH,
            )
        pl.semaphore_wait(barrier_sem, 3)


        seg_lo = {}
        halves = {}

        def rs_descr(p, k):
            base, rows, dims = _PARTS[p]
            d = dims[k]
            part = partners[d]
            half = coords[d]
            halves[(p, k)] = half
            l2 = rows >> (k + 1)
            lo = seg_lo[p]
            send_lo = lo + (1 - half) * l2
            off = base + (0 if k == 0 else (rows >> 1) if k == 1 else 3 * (rows >> 2))
            src = x_ref if k == 0 else out_ref
            return pltpu.make_async_remote_copy(
                src_ref=src.at[pl.ds(send_lo, l2)],
                dst_ref=comm_ref.at[pl.ds(off, l2)],
                send_sem=rs_send.at[p, k],
                recv_sem=rs_recv.at[p, k],
                device_id=(part,),
                device_id_type=pl.DeviceIdType.MESH,
            ), off, l2

        def ag_descr(p, k):
            base, rows, dims = _PARTS[p]
            d = dims[2 - k]
            part = partners[d]
            l = rows >> (3 - k)
            lo = seg_lo[p]
            return pltpu.make_async_remote_copy(
                src_ref=out_ref.at[pl.ds(lo, l)],
                dst_ref=out_ref.at[pl.ds(lo, l)],
                send_sem=ag_send.at[p, k],
                recv_sem=ag_recv.at[p, k],
                device_id=(part,),
                device_id_type=pl.DeviceIdType.MESH,
            )

        for p in range(_NP):
            seg_lo[p] = jnp.int32(_PARTS[p][0])
        inflight = {}
        for p in range(_NP):
            rdma, off, l2 = rs_descr(p, 0)
            rdma.start()
            inflight[p] = (rdma, off, l2)

        ag_inflight = {}
        for k in range(3):
            for p in range(_NP):
                rdma, off, l2 = inflight[p]
                rdma.wait()
                half = halves[(p, k)]
                keep_lo = seg_lo[p] + half * l2
                idx = pl.ds(keep_lo, l2)
                acc_src = x_ref if k == 0 else out_ref
                out_ref[idx] = acc_src[idx] + comm_ref[pl.ds(off, l2)]
                seg_lo[p] = keep_lo
                if k < 2:
                    nxt, noff, nl2 = rs_descr(p, k + 1)
                    nxt.start()
                    inflight[p] = (nxt, noff, nl2)
                else:
                    ag = ag_descr(p, 0)
                    ag.start()
                    ag_inflight[p] = ag

        for k in range(3):
            for p in range(_NP):
                rows = _PARTS[p][1]
                l = rows >> (3 - k)
                ag_inflight[p].wait()
                seg_lo[p] = seg_lo[p] - halves[(p, 2 - k)] * l
                if k < 2:
                    ag = ag_descr(p, k + 1)
                    ag.start()
                    ag_inflight[p] = ag

        @functools.partial(
            pl.run_scoped, second_barrier=pltpu.SemaphoreType.REGULAR
        )
        def _(second_barrier):
            for d in ("x", "y", "z"):
                pl.semaphore_signal(
                    second_barrier, inc=1,
                    device_id=(partners[d],),
                    device_id_type=pl.DeviceIdType.MESH,
                )
            pl.semaphore_wait(second_barrier, 3)

    return pl.pallas_call(
        body,
        out_shape=jax.ShapeDtypeStruct((m_per, n), x.dtype),
        in_specs=[pl.BlockSpec(memory_space=pltpu.VMEM)],
        out_specs=pl.BlockSpec(memory_space=pltpu.VMEM),
        scratch_shapes=[
            pltpu.VMEM((m_per, n), x.dtype),
            pltpu.SemaphoreType.DMA((_NP, 3)),
            pltpu.SemaphoreType.DMA((_NP, 3)),
            pltpu.SemaphoreType.DMA((_NP, 3)),
            pltpu.SemaphoreType.DMA((_NP, 3)),
        ],
        compiler_params=pltpu.CompilerParams(collective_id=0),
    )(x)


# device time: 124439 ns/iter; 2.8649x vs baseline; 1.0430x over previous
import functools

import jax
import jax.numpy as jnp
from jax import lax
from jax.experimental import pallas as pl
from jax.experimental.pallas import tpu as pltpu

N_DEV = 8

_MASK = {"x": 1, "y": 3, "z": 4}

_PARTS = (
    (0, 704, ("x", "y", "z")),
    (704, 704, ("x", "y", "z")),
    (1408, 704, ("y", "z", "x")),
    (2112, 640, ("y", "z", "x")),
    (2752, 704, ("z", "x", "y")),
    (3456, 640, ("z", "x", "y")),
)
_NP = len(_PARTS)


def kernel(x):
    m_per, n = x.shape
    assert m_per == 4096

    def body(x_ref, out_ref, comm_ref, rs_send, rs_recv, ag_send, ag_recv):
        me = lax.axis_index("i")
        partners = {d: me ^ mask for d, mask in _MASK.items()}
        coords = {
            "x": (me ^ (me >> 1)) & 1,
            "y": (me >> 1) & 1,
            "z": (me >> 2) & 1,
        }

        barrier_sem = pltpu.get_barrier_semaphore()
        for d in ("x", "y", "z"):
            pl.semaphore_signal(
                barrier_sem, inc=1,
                device_id=(partners[d],), device_id_type=pl.DeviceIdType.MESH,
            )
        pl.semaphore_wait(barrier_sem, 3)


        seg_lo = {}
        halves = {}

        def rs_descr(p, k):
            base, rows, dims = _PARTS[p]
            d = dims[k]
            part = partners[d]
            half = coords[d]
            halves[(p, k)] = half
            l2 = rows >> (k + 1)
            lo = seg_lo[p]
            send_lo = lo + (1 - half) * l2
            off = base + (0 if k == 0 else (rows >> 1) if k == 1 else 3 * (rows >> 2))
            src = x_ref if k == 0 else out_ref
            return pltpu.make_async_remote_copy(
                src_ref=src.at[pl.ds(send_lo, l2)],
                dst_ref=comm_ref.at[pl.ds(off, l2)],
                send_sem=rs_send.at[p, k],
                recv_sem=rs_recv.at[p, k],
                device_id=(part,),
                device_id_type=pl.DeviceIdType.MESH,
            ), off, l2

        def ag_descr(p, k):
            base, rows, dims = _PARTS[p]
            d = dims[2 - k]
            part = partners[d]
            l = rows >> (3 - k)
            lo = seg_lo[p]
            return pltpu.make_async_remote_copy(
                src_ref=out_ref.at[pl.ds(lo, l)],
                dst_ref=out_ref.at[pl.ds(lo, l)],
                send_sem=ag_send.at[p, k],
                recv_sem=ag_recv.at[p, k],
                device_id=(part,),
                device_id_type=pl.DeviceIdType.MESH,
            )

        for p in range(_NP):
            seg_lo[p] = jnp.int32(_PARTS[p][0])
        inflight = {}
        for p in range(_NP):
            rdma, off, l2 = rs_descr(p, 0)
            rdma.start()
            inflight[p] = (rdma, off, l2)

        _ORDER = (0, 2, 4, 1, 3, 5)
        ag_inflight = {}
        for k in range(3):
            for p in _ORDER:
                rdma, off, l2 = inflight[p]
                rdma.wait()
                half = halves[(p, k)]
                keep_lo = seg_lo[p] + half * l2
                acc_src = x_ref if k == 0 else out_ref
                seg_lo[p] = keep_lo
                if k < 2:
                    l4 = l2 >> 1
                    nxt_half = coords[_PARTS[p][2][k + 1]]
                    q1 = (1 - nxt_half) * l4
                    idx = pl.ds(keep_lo + q1, l4)
                    out_ref[idx] = acc_src[idx] + comm_ref[pl.ds(off + q1, l4)]
                    nxt, noff, nl2 = rs_descr(p, k + 1)
                    nxt.start()
                    inflight[p] = (nxt, noff, nl2)
                    q2 = l4 - q1
                    idx = pl.ds(keep_lo + q2, l4)
                    out_ref[idx] = acc_src[idx] + comm_ref[pl.ds(off + q2, l4)]
                else:
                    idx = pl.ds(keep_lo, l2)
                    out_ref[idx] = acc_src[idx] + comm_ref[pl.ds(off, l2)]
                    ag = ag_descr(p, 0)
                    ag.start()
                    ag_inflight[p] = ag

        for k in range(3):
            for p in _ORDER:
                rows = _PARTS[p][1]
                l = rows >> (3 - k)
                ag_inflight[p].wait()
                seg_lo[p] = seg_lo[p] - halves[(p, 2 - k)] * l
                if k < 2:
                    ag = ag_descr(p, k + 1)
                    ag.start()
                    ag_inflight[p] = ag

        @functools.partial(
            pl.run_scoped, second_barrier=pltpu.SemaphoreType.REGULAR
        )
        def _(second_barrier):
            for d in ("x", "y", "z"):
                pl.semaphore_signal(
                    second_barrier, inc=1,
                    device_id=(partners[d],),
                    device_id_type=pl.DeviceIdType.MESH,
                )
            pl.semaphore_wait(second_barrier, 3)

    return pl.pallas_call(
        body,
        out_shape=jax.ShapeDtypeStruct((m_per, n), x.dtype),
        in_specs=[pl.BlockSpec(memory_space=pltpu.VMEM)],
        out_specs=pl.BlockSpec(memory_space=pltpu.VMEM),
        scratch_shapes=[
            pltpu.VMEM((m_per, n), x.dtype),
            pltpu.SemaphoreType.DMA((_NP, 3)),
            pltpu.SemaphoreType.DMA((_NP, 3)),
            pltpu.SemaphoreType.DMA((_NP, 3)),
            pltpu.SemaphoreType.DMA((_NP, 3)),
        ],
        compiler_params=pltpu.CompilerParams(collective_id=0),
    )(x)
